# baseline (speedup 1.0000x reference)
import functools

import jax
import jax.numpy as jnp
from jax import lax
from jax.experimental import pallas as pl
from jax.experimental.pallas import tpu as pltpu
from jax.experimental.pallas import tpu_sc as plsc

N_NODES = 10000
N_PAD = 10240
BN = 512
NB = N_PAD // BN
HID = 128
SEQ = 20
INP = 16
NC, NS, LANES = 2, 16, 16
CHUNK = 2048
E_IN = 320000
E_TOT = E_IN + N_NODES
NCHUNK = -(-E_TOT // CHUNK)
if NCHUNK % 2:
    NCHUNK += 1
E2 = NCHUNK * CHUNK
DROWS = N_PAD // HID


def _lstm_body(x_ref, wih_ref, whh_ref, b_ref, out_ref):
    HB = BN // 2
    hs = [jnp.zeros((HB, HID), jnp.float32) for _ in range(2)]
    cs = [jnp.zeros((HB, HID), jnp.float32) for _ in range(2)]
    b = b_ref[...]
    wih = wih_ref[...]
    whh = whh_ref[...]
    for t in range(SEQ):
        for p in range(2):
            xt = x_ref[p * HB:(p + 1) * HB, t * INP:(t + 1) * INP]
            g = (lax.dot_general(xt, wih, (((1,), (1,)), ((), ())),
                                 preferred_element_type=jnp.float32)
                 + lax.dot_general(hs[p], whh, (((1,), (1,)), ((), ())),
                                   preferred_element_type=jnp.float32)
                 + b)
            i = jax.nn.sigmoid(g[:, 0:HID])
            f = jax.nn.sigmoid(g[:, HID:2 * HID])
            gg = jnp.tanh(g[:, 2 * HID:3 * HID])
            o = jax.nn.sigmoid(g[:, 3 * HID:4 * HID])
            cs[p] = f * cs[p] + i * gg
            hs[p] = o * jnp.tanh(cs[p])
    out_ref[0:HB, :] = hs[0]
    out_ref[HB:BN, :] = hs[1]


def _run_lstm(xp, wih, whh, bias):
    return pl.pallas_call(
        _lstm_body,
        grid=(NB,),
        in_specs=[
            pl.BlockSpec((BN, SEQ * INP), lambda j: (j, 0)),
            pl.BlockSpec((4 * HID, INP), lambda j: (0, 0)),
            pl.BlockSpec((4 * HID, HID), lambda j: (0, 0)),
            pl.BlockSpec((1, 4 * HID), lambda j: (0, 0)),
        ],
        out_specs=pl.BlockSpec((BN, HID), lambda j: (j, 0)),
        out_shape=jax.ShapeDtypeStruct((N_PAD, HID), jnp.float32),
    )(xp, wih, whh, bias)


def _proj_body(tx_ref, ghW_ref, gcW_ref, att_ref,
               xlTh_ref, xlTc_ref, avec_ref, mvec_ref):
    j = pl.program_id(0)
    tx = tx_ref[...]
    att = att_ref[...]
    xlTh = lax.dot_general(ghW_ref[...], tx, (((1,), (1,)), ((), ())),
                           preferred_element_type=jnp.float32)
    xlTc = lax.dot_general(gcW_ref[...], tx, (((1,), (1,)), ((), ())),
                           preferred_element_type=jnp.float32)
    xlTh_ref[...] = xlTh
    xlTc_ref[...] = xlTc
    ah = lax.dot_general(att[0:2, :], xlTh, (((1,), (0,)), ((), ())),
                         preferred_element_type=jnp.float32)
    ac = lax.dot_general(att[2:4, :], xlTc, (((1,), (0,)), ((), ())),
                         preferred_element_type=jnp.float32)
    av = jnp.concatenate([ah, ac, jnp.zeros((4, BN), jnp.float32)], axis=0)
    avec_ref[...] = av
    mb = jnp.broadcast_to(jnp.max(av, axis=1, keepdims=True), (8, HID))

    @pl.when(j == 0)
    def _():
        mvec_ref[...] = mb

    @pl.when(j > 0)
    def _():
        mvec_ref[...] = jnp.maximum(mvec_ref[...], mb)


def _run_proj(tx, ghW, gcW, att):
    return pl.pallas_call(
        _proj_body,
        grid=(NB,),
        in_specs=[
            pl.BlockSpec((BN, HID), lambda j: (j, 0)),
            pl.BlockSpec((HID, HID), lambda j: (0, 0)),
            pl.BlockSpec((HID, HID), lambda j: (0, 0)),
            pl.BlockSpec((8, HID), lambda j: (0, 0)),
        ],
        out_specs=[
            pl.BlockSpec((HID, BN), lambda j: (0, j)),
            pl.BlockSpec((HID, BN), lambda j: (0, j)),
            pl.BlockSpec((8, BN), lambda j: (0, j)),
            pl.BlockSpec((8, HID), lambda j: (0, 0)),
        ],
        out_shape=[
            jax.ShapeDtypeStruct((HID, N_PAD), jnp.float32),
            jax.ShapeDtypeStruct((HID, N_PAD), jnp.float32),
            jax.ShapeDtypeStruct((8, N_PAD), jnp.float32),
            jax.ShapeDtypeStruct((8, HID), jnp.float32),
        ],
    )(tx, ghW, gcW, att)


def _projflat_body(tx_ref, ghW_ref, gcW_ref, oh_ref, oc_ref):
    tx = tx_ref[...]
    rh = lax.dot_general(ghW_ref[...], tx, (((1,), (1,)), ((), ())),
                         preferred_element_type=jnp.float32)
    oh_ref[...] = rh.reshape(8 * N_PAD)
    rc = lax.dot_general(gcW_ref[...], tx, (((1,), (1,)), ((), ())),
                         preferred_element_type=jnp.float32)
    oc_ref[...] = rc.reshape(8 * N_PAD)


def _run_projflat(tx, ghW, gcW):
    return pl.pallas_call(
        _projflat_body,
        grid=(HID // 8,),
        in_specs=[
            pl.BlockSpec((N_PAD, HID), lambda j: (0, 0)),
            pl.BlockSpec((8, HID), lambda j: (j, 0)),
            pl.BlockSpec((8, HID), lambda j: (j, 0)),
        ],
        out_specs=[pl.BlockSpec((8 * N_PAD,), lambda j: (j,)),
                   pl.BlockSpec((8 * N_PAD,), lambda j: (j,))],
        out_shape=[jax.ShapeDtypeStruct((HID * N_PAD,), jnp.float32),
                   jax.ShapeDtypeStruct((HID * N_PAD,), jnp.float32)],
    )(tx, ghW, gcW)


def _gat_sc_body(asrc_hbm, adst_hbm, xlT_hbm, src_hbm, dst_hbm, m_hbm,
                 outT_hbm, coef_hbm,
                 asrc_v, adst_v, den_v, xl0_v, xl1_v, xl2_v, xl3_v,
                 ot0_v, ot1_v, ot2_v, ot3_v, sbuf, dbuf, cbuf,
                 m_v, idx_v, den_sh, sem):
    c_id = lax.axis_index("c")
    s_id = lax.axis_index("s")
    wid = s_id * NC + c_id
    coef_base = c_id * E2
    xls = (xl0_v, xl1_v, xl2_v, xl3_v)
    ots = (ot0_v, ot1_v, ot2_v, ot3_v)

    pltpu.sync_copy(asrc_hbm, asrc_v)
    pltpu.sync_copy(adst_hbm, adst_v)
    for r in range(4):
        pltpu.sync_copy(xlT_hbm.at[pl.ds((wid * 4 + r) * N_PAD, N_PAD)],
                        xls[r])
    pltpu.sync_copy(m_hbm, m_v)
    m = m_v[...]

    zeros16 = jnp.zeros((LANES,), jnp.float32)
    iota16 = lax.iota(jnp.int32, LANES)

    def zbody(i, _):
        for r in range(4):
            ots[r][pl.ds(i * LANES, LANES)] = zeros16
        return 0
    lax.fori_loop(0, N_PAD // LANES, zbody, 0)

    def zdbody(i, _):
        for r in range(HID // LANES):
            den_v[i, pl.ds(r * LANES, LANES)] = zeros16
        return 0
    lax.fori_loop(0, DROWS, zdbody, 0)

    for i in range(DROWS // LANES):
        idx_v[pl.ds(i * LANES, LANES)] = iota16 + (i * LANES)

    @pl.when(s_id == 0)
    def _():
        pltpu.sync_copy(den_v, den_sh)
    plsc.subcore_barrier()

    def gather_alpha(sv, dv):
        a1 = plsc.load_gather(asrc_v, [sv])
        a2 = plsc.load_gather(adst_v, [dv])
        z = a1 + a2
        alpha = jnp.where(z > 0, z, 0.2 * z)
        return jnp.exp(alpha - m)

    def own_chunks(chunk_body):
        def a_body(k, _):
            ca = s_id + k * NS

            @pl.when(ca < NCHUNK)
            def _():
                pltpu.sync_copy(src_hbm.at[pl.ds(ca * CHUNK, CHUNK)],
                                sbuf.at[0])
                pltpu.sync_copy(dst_hbm.at[pl.ds(ca * CHUNK, CHUNK)],
                                dbuf.at[0])
                chunk_body(ca)
            return 0
        lax.fori_loop(0, -(-NCHUNK // NS), a_body, 0)

    def passA(ca):
        @plsc.parallel_loop(0, CHUNK, LANES, unroll=4)
        def g_body(g):
            sv = sbuf[0, pl.ds(g, LANES)]
            dv = dbuf[0, pl.ds(g, LANES)]
            ex = gather_alpha(sv, dv)
            plsc.addupdate_scatter(den_v, [dv >> 7, dv & 127], ex)
    own_chunks(passA)

    plsc.subcore_barrier()
    pltpu.sync_copy(den_v, den_sh.at[idx_v], add=True)
    plsc.subcore_barrier()
    pltpu.sync_copy(den_sh, den_v)

    def passA2(ca):
        @plsc.parallel_loop(0, CHUNK, LANES, unroll=4)
        def g_body(g):
            sv = sbuf[0, pl.ds(g, LANES)]
            dv = dbuf[0, pl.ds(g, LANES)]
            ex = gather_alpha(sv, dv)
            den = plsc.load_gather(den_v, [dv >> 7, dv & 127])
            cbuf[0, pl.ds(g, LANES)] = ex / jnp.maximum(den, 1e-30)
        pltpu.sync_copy(cbuf.at[0],
                        coef_hbm.at[pl.ds(coef_base + ca * CHUNK, CHUNK)])
    own_chunks(passA2)

    plsc.subcore_barrier()

    def start_fetch(cb, slot):
        pltpu.async_copy(src_hbm.at[pl.ds(cb * CHUNK, CHUNK)],
                         sbuf.at[slot], sem.at[slot])
        pltpu.async_copy(dst_hbm.at[pl.ds(cb * CHUNK, CHUNK)],
                         dbuf.at[slot], sem.at[slot])
        pltpu.async_copy(coef_hbm.at[pl.ds(coef_base + cb * CHUNK, CHUNK)],
                         cbuf.at[slot], sem.at[slot])

    def wait_fetch(slot):
        pltpu.make_async_copy(src_hbm.at[pl.ds(0, CHUNK)],
                              sbuf.at[slot], sem.at[slot]).wait()
        pltpu.make_async_copy(dst_hbm.at[pl.ds(0, CHUNK)],
                              dbuf.at[slot], sem.at[slot]).wait()
        pltpu.make_async_copy(coef_hbm.at[pl.ds(0, CHUNK)],
                              cbuf.at[slot], sem.at[slot]).wait()

    def compute_chunk(slot):
        @plsc.parallel_loop(0, CHUNK, LANES, unroll=4)
        def g_body(g):
            sv = sbuf[slot, pl.ds(g, LANES)]
            dv = dbuf[slot, pl.ds(g, LANES)]
            coef = cbuf[slot, pl.ds(g, LANES)]
            for r in range(4):
                vals = plsc.load_gather(xls[r], [sv])
                plsc.addupdate_scatter(ots[r], [dv], vals * coef)

    start_fetch(0, 0)

    def outer(k, _):
        cb0 = 2 * k

        @pl.when(cb0 + 1 < NCHUNK)
        def _():
            start_fetch(cb0 + 1, 1)
        wait_fetch(0)
        compute_chunk(0)

        @pl.when(cb0 + 2 < NCHUNK)
        def _():
            start_fetch(cb0 + 2, 0)
        wait_fetch(1)
        compute_chunk(1)
        return 0
    lax.fori_loop(0, NCHUNK // 2, outer, 0)

    for r in range(4):
        pltpu.sync_copy(ots[r],
                        outT_hbm.at[pl.ds((wid * 4 + r) * N_PAD, N_PAD)])


@functools.lru_cache(maxsize=1)
def _make_gat_sc():
    mesh = plsc.VectorSubcoreMesh(core_axis_name="c", subcore_axis_name="s",
                                  num_cores=NC, num_subcores=NS)
    return pl.kernel(
        _gat_sc_body,
        out_type=[jax.ShapeDtypeStruct((HID * N_PAD,), jnp.float32),
                  jax.ShapeDtypeStruct((NC * E2,), jnp.float32)],
        mesh=mesh,
        scratch_types=[
            pltpu.VMEM((N_PAD,), jnp.float32),
            pltpu.VMEM((N_PAD,), jnp.float32),
            pltpu.VMEM((DROWS, HID), jnp.float32),
            pltpu.VMEM((N_PAD,), jnp.float32),
            pltpu.VMEM((N_PAD,), jnp.float32),
            pltpu.VMEM((N_PAD,), jnp.float32),
            pltpu.VMEM((N_PAD,), jnp.float32),
            pltpu.VMEM((N_PAD,), jnp.float32),
            pltpu.VMEM((N_PAD,), jnp.float32),
            pltpu.VMEM((N_PAD,), jnp.float32),
            pltpu.VMEM((N_PAD,), jnp.float32),
            pltpu.VMEM((2, CHUNK), jnp.int32),
            pltpu.VMEM((2, CHUNK), jnp.int32),
            pltpu.VMEM((2, CHUNK), jnp.float32),
            pltpu.VMEM((LANES,), jnp.float32),
            pltpu.VMEM((DROWS,), jnp.int32),
            pltpu.VMEM_SHARED((DROWS, HID), jnp.float32),
            pltpu.SemaphoreType.DMA((2,)),
        ],
        compiler_params=pltpu.CompilerParams(needs_layout_passes=False),
    )


def _dec_body(tx_ref, rTh_ref, rTc_ref, w_ref, ghb_ref, gcb_ref, db_ref,
              out_ref):
    w = w_ref[...]
    w1, w2, w3 = w[:, 0:HID], w[:, HID:2 * HID], w[:, 2 * HID:3 * HID]
    y = lax.dot_general(w1, tx_ref[...], (((1,), (1,)), ((), ())),
                        preferred_element_type=jnp.float32)
    y = y + lax.dot_general(w2, rTh_ref[...], (((1,), (0,)), ((), ())),
                            preferred_element_type=jnp.float32)
    y = y + lax.dot_general(w3, rTc_ref[...], (((1,), (0,)), ((), ())),
                            preferred_element_type=jnp.float32)
    cb = (jnp.sum(w2 * ghb_ref[...]) + jnp.sum(w3 * gcb_ref[...])
          + db_ref[0, 0])
    out_ref[...] = y + cb


def _run_dec(tx, rTh, rTc, w, ghb, gcb, db):
    return pl.pallas_call(
        _dec_body,
        grid=(NB,),
        in_specs=[
            pl.BlockSpec((BN, HID), lambda j: (j, 0)),
            pl.BlockSpec((HID, BN), lambda j: (0, j)),
            pl.BlockSpec((HID, BN), lambda j: (0, j)),
            pl.BlockSpec((1, 3 * HID), lambda j: (0, 0)),
            pl.BlockSpec((1, HID), lambda j: (0, 0)),
            pl.BlockSpec((1, HID), lambda j: (0, 0)),
            pl.BlockSpec((1, 1), lambda j: (0, 0)),
        ],
        out_specs=pl.BlockSpec((1, BN), lambda j: (0, j)),
        out_shape=jax.ShapeDtypeStruct((1, N_PAD), jnp.float32),
    )(tx, rTh, rTc, w, ghb, gcb, db)


def _prep_edges(ei):
    s, d = ei[0].astype(jnp.int32), ei[1].astype(jnp.int32)
    is_loop = s == d
    pad = jnp.int32(N_NODES)
    s = jnp.where(is_loop, pad, s)
    d = jnp.where(is_loop, pad, d)
    ar = jnp.arange(N_NODES, dtype=jnp.int32)
    npad_e = E2 - E_TOT
    fill = jnp.full((npad_e,), pad, jnp.int32)
    src = jnp.concatenate([s, ar, fill])
    dst = jnp.concatenate([d, ar, fill])
    return src, dst


def kernel(x, edge_index_hetero, edge_index,
           lstm_W_ih, lstm_W_hh, lstm_b_ih, lstm_b_hh,
           gh_W, gh_att_src, gh_att_dst, gh_bias,
           gc_W, gc_att_src, gc_att_dst, gc_bias,
           dec_W, dec_b):
    xp = jnp.pad(x.reshape(N_NODES, SEQ * INP),
                 ((0, N_PAD - N_NODES), (0, 0)))
    bias = (lstm_b_ih + lstm_b_hh).reshape(1, 4 * HID)
    tx = _run_lstm(xp, lstm_W_ih, lstm_W_hh, bias)

    att = jnp.concatenate([
        gh_att_src.reshape(1, HID), gh_att_dst.reshape(1, HID),
        gc_att_src.reshape(1, HID), gc_att_dst.reshape(1, HID),
        jnp.zeros((4, HID), jnp.float32)], axis=0)
    _, _, avec, mvec = _run_proj(tx, gh_W, gc_W, att)
    xlTh_f, xlTc_f = _run_projflat(tx, gh_W, gc_W)

    m_h = jnp.full((LANES,), mvec[0, 0] + mvec[1, 0], jnp.float32)
    m_c = jnp.full((LANES,), mvec[2, 0] + mvec[3, 0], jnp.float32)

    src_h, dst_h = _prep_edges(edge_index_hetero)
    src_c, dst_c = _prep_edges(edge_index)

    gat = _make_gat_sc()
    rTh, _ = gat(avec[0], avec[1], xlTh_f, src_h, dst_h, m_h)
    rTc, _ = gat(avec[2], avec[3], xlTc_f, src_c, dst_c, m_c)
    rTh = rTh.reshape(HID, N_PAD)
    rTc = rTc.reshape(HID, N_PAD)

    out = _run_dec(tx, rTh, rTc, dec_W.reshape(1, 3 * HID),
                   gh_bias.reshape(1, HID), gc_bias.reshape(1, HID),
                   dec_b.reshape(1, 1))
    return out.reshape(N_PAD, 1)[:N_NODES]


# speedup vs baseline: 1.0223x; 1.0223x over previous
import functools

import jax
import jax.numpy as jnp
from jax import lax
from jax.experimental import pallas as pl
from jax.experimental.pallas import tpu as pltpu
from jax.experimental.pallas import tpu_sc as plsc

N_NODES = 10000
N_PAD = 10240
BN = 512
NB = N_PAD // BN
HID = 128
SEQ = 20
INP = 16
NC, NS, LANES = 2, 16, 16
CHUNK = 2048
E_IN = 320000
E_TOT = E_IN + N_NODES
NCHUNK = -(-E_TOT // CHUNK)
if NCHUNK % 2:
    NCHUNK += 1
E2 = NCHUNK * CHUNK
DROWS = N_PAD // HID


def _lstm_body(x_ref, wih_ref, whh_ref, b_ref, out_ref):
    HB = BN // 2
    hs = [jnp.zeros((HB, HID), jnp.float32) for _ in range(2)]
    cs = [jnp.zeros((HB, HID), jnp.float32) for _ in range(2)]
    b = b_ref[...]
    wih = wih_ref[...]
    whh = whh_ref[...]
    for t in range(SEQ):
        for p in range(2):
            xt = x_ref[p * HB:(p + 1) * HB, t * INP:(t + 1) * INP]
            g = (lax.dot_general(xt, wih, (((1,), (1,)), ((), ())),
                                 preferred_element_type=jnp.float32)
                 + lax.dot_general(hs[p], whh, (((1,), (1,)), ((), ())),
                                   preferred_element_type=jnp.float32)
                 + b)
            i = jax.nn.sigmoid(g[:, 0:HID])
            f = jax.nn.sigmoid(g[:, HID:2 * HID])
            gg = jnp.tanh(g[:, 2 * HID:3 * HID])
            o = jax.nn.sigmoid(g[:, 3 * HID:4 * HID])
            cs[p] = f * cs[p] + i * gg
            hs[p] = o * jnp.tanh(cs[p])
    out_ref[0:HB, :] = hs[0]
    out_ref[HB:BN, :] = hs[1]


def _run_lstm(xp, wih, whh, bias):
    return pl.pallas_call(
        _lstm_body,
        grid=(NB,),
        in_specs=[
            pl.BlockSpec((BN, SEQ * INP), lambda j: (j, 0)),
            pl.BlockSpec((4 * HID, INP), lambda j: (0, 0)),
            pl.BlockSpec((4 * HID, HID), lambda j: (0, 0)),
            pl.BlockSpec((1, 4 * HID), lambda j: (0, 0)),
        ],
        out_specs=pl.BlockSpec((BN, HID), lambda j: (j, 0)),
        out_shape=jax.ShapeDtypeStruct((N_PAD, HID), jnp.float32),
    )(xp, wih, whh, bias)


def _proj_body(tx_ref, ghW_ref, gcW_ref, att_ref,
               xlTh_ref, xlTc_ref, avec_ref, mvec_ref):
    j = pl.program_id(0)
    tx = tx_ref[...]
    att = att_ref[...]
    xlTh = lax.dot_general(ghW_ref[...], tx, (((1,), (1,)), ((), ())),
                           preferred_element_type=jnp.float32)
    xlTc = lax.dot_general(gcW_ref[...], tx, (((1,), (1,)), ((), ())),
                           preferred_element_type=jnp.float32)
    xlTh_ref[...] = xlTh
    xlTc_ref[...] = xlTc
    ah = lax.dot_general(att[0:2, :], xlTh, (((1,), (0,)), ((), ())),
                         preferred_element_type=jnp.float32)
    ac = lax.dot_general(att[2:4, :], xlTc, (((1,), (0,)), ((), ())),
                         preferred_element_type=jnp.float32)
    av = jnp.concatenate([ah, ac, jnp.zeros((4, BN), jnp.float32)], axis=0)
    avec_ref[...] = av
    mb = jnp.broadcast_to(jnp.max(av, axis=1, keepdims=True), (8, HID))

    @pl.when(j == 0)
    def _():
        mvec_ref[...] = mb

    @pl.when(j > 0)
    def _():
        mvec_ref[...] = jnp.maximum(mvec_ref[...], mb)


def _run_proj(tx, ghW, gcW, att):
    return pl.pallas_call(
        _proj_body,
        grid=(NB,),
        in_specs=[
            pl.BlockSpec((BN, HID), lambda j: (j, 0)),
            pl.BlockSpec((HID, HID), lambda j: (0, 0)),
            pl.BlockSpec((HID, HID), lambda j: (0, 0)),
            pl.BlockSpec((8, HID), lambda j: (0, 0)),
        ],
        out_specs=[
            pl.BlockSpec((HID, BN), lambda j: (0, j)),
            pl.BlockSpec((HID, BN), lambda j: (0, j)),
            pl.BlockSpec((8, BN), lambda j: (0, j)),
            pl.BlockSpec((8, HID), lambda j: (0, 0)),
        ],
        out_shape=[
            jax.ShapeDtypeStruct((HID, N_PAD), jnp.float32),
            jax.ShapeDtypeStruct((HID, N_PAD), jnp.float32),
            jax.ShapeDtypeStruct((8, N_PAD), jnp.float32),
            jax.ShapeDtypeStruct((8, HID), jnp.float32),
        ],
    )(tx, ghW, gcW, att)


def _gat_sc_body(asrc_hbm, adst_hbm, xlT_hbm, src_hbm, dst_hbm, m_hbm,
                 outT_hbm, coef_hbm,
                 asrc_v, adst_v, den_v, xl0_v, xl1_v, xl2_v, xl3_v,
                 ot0_v, ot1_v, ot2_v, ot3_v, sbuf, dbuf, cbuf,
                 m_v, idx_v, den_sh, sem):
    c_id = lax.axis_index("c")
    s_id = lax.axis_index("s")
    wid = s_id * NC + c_id
    coef_base = c_id * E2
    xls = (xl0_v, xl1_v, xl2_v, xl3_v)
    ots = (ot0_v, ot1_v, ot2_v, ot3_v)

    pltpu.sync_copy(asrc_hbm, asrc_v)
    pltpu.sync_copy(adst_hbm, adst_v)
    for r in range(4):
        pltpu.sync_copy(xlT_hbm.at[pl.ds((wid * 4 + r) * N_PAD, N_PAD)],
                        xls[r])
    pltpu.sync_copy(m_hbm, m_v)
    m = m_v[...]

    zeros16 = jnp.zeros((LANES,), jnp.float32)
    iota16 = lax.iota(jnp.int32, LANES)

    def zbody(i, _):
        for r in range(4):
            ots[r][pl.ds(i * LANES, LANES)] = zeros16
        return 0
    lax.fori_loop(0, N_PAD // LANES, zbody, 0)

    def zdbody(i, _):
        for r in range(HID // LANES):
            den_v[i, pl.ds(r * LANES, LANES)] = zeros16
        return 0
    lax.fori_loop(0, DROWS, zdbody, 0)

    for i in range(DROWS // LANES):
        idx_v[pl.ds(i * LANES, LANES)] = iota16 + (i * LANES)

    @pl.when(s_id == 0)
    def _():
        pltpu.sync_copy(den_v, den_sh)
    plsc.subcore_barrier()

    def gather_alpha(sv, dv):
        a1 = plsc.load_gather(asrc_v, [sv])
        a2 = plsc.load_gather(adst_v, [dv])
        z = a1 + a2
        alpha = jnp.where(z > 0, z, 0.2 * z)
        return jnp.exp(alpha - m)

    def own_chunks(chunk_body):
        def a_body(k, _):
            ca = s_id + k * NS

            @pl.when(ca < NCHUNK)
            def _():
                pltpu.sync_copy(src_hbm.at[pl.ds(ca * CHUNK, CHUNK)],
                                sbuf.at[0])
                pltpu.sync_copy(dst_hbm.at[pl.ds(ca * CHUNK, CHUNK)],
                                dbuf.at[0])
                chunk_body(ca)
            return 0
        lax.fori_loop(0, -(-NCHUNK // NS), a_body, 0)

    def passA(ca):
        @plsc.parallel_loop(0, CHUNK, LANES, unroll=4)
        def g_body(g):
            sv = sbuf[0, pl.ds(g, LANES)]
            dv = dbuf[0, pl.ds(g, LANES)]
            ex = gather_alpha(sv, dv)
            plsc.addupdate_scatter(den_v, [dv >> 7, dv & 127], ex)
    own_chunks(passA)

    plsc.subcore_barrier()
    pltpu.sync_copy(den_v, den_sh.at[idx_v], add=True)
    plsc.subcore_barrier()
    pltpu.sync_copy(den_sh, den_v)

    def passA2(ca):
        @plsc.parallel_loop(0, CHUNK, LANES, unroll=4)
        def g_body(g):
            sv = sbuf[0, pl.ds(g, LANES)]
            dv = dbuf[0, pl.ds(g, LANES)]
            ex = gather_alpha(sv, dv)
            den = plsc.load_gather(den_v, [dv >> 7, dv & 127])
            cbuf[0, pl.ds(g, LANES)] = ex / jnp.maximum(den, 1e-30)
        pltpu.sync_copy(cbuf.at[0],
                        coef_hbm.at[pl.ds(coef_base + ca * CHUNK, CHUNK)])
    own_chunks(passA2)

    plsc.subcore_barrier()

    def start_fetch(cb, slot):
        pltpu.async_copy(src_hbm.at[pl.ds(cb * CHUNK, CHUNK)],
                         sbuf.at[slot], sem.at[slot])
        pltpu.async_copy(dst_hbm.at[pl.ds(cb * CHUNK, CHUNK)],
                         dbuf.at[slot], sem.at[slot])
        pltpu.async_copy(coef_hbm.at[pl.ds(coef_base + cb * CHUNK, CHUNK)],
                         cbuf.at[slot], sem.at[slot])

    def wait_fetch(slot):
        pltpu.make_async_copy(src_hbm.at[pl.ds(0, CHUNK)],
                              sbuf.at[slot], sem.at[slot]).wait()
        pltpu.make_async_copy(dst_hbm.at[pl.ds(0, CHUNK)],
                              dbuf.at[slot], sem.at[slot]).wait()
        pltpu.make_async_copy(coef_hbm.at[pl.ds(0, CHUNK)],
                              cbuf.at[slot], sem.at[slot]).wait()

    def compute_chunk(slot):
        @plsc.parallel_loop(0, CHUNK, LANES, unroll=4)
        def g_body(g):
            sv = sbuf[slot, pl.ds(g, LANES)]
            dv = dbuf[slot, pl.ds(g, LANES)]
            coef = cbuf[slot, pl.ds(g, LANES)]
            for r in range(4):
                vals = plsc.load_gather(xls[r], [sv])
                plsc.addupdate_scatter(ots[r], [dv], vals * coef)

    start_fetch(0, 0)

    def outer(k, _):
        cb0 = 2 * k

        @pl.when(cb0 + 1 < NCHUNK)
        def _():
            start_fetch(cb0 + 1, 1)
        wait_fetch(0)
        compute_chunk(0)

        @pl.when(cb0 + 2 < NCHUNK)
        def _():
            start_fetch(cb0 + 2, 0)
        wait_fetch(1)
        compute_chunk(1)
        return 0
    lax.fori_loop(0, NCHUNK // 2, outer, 0)

    for r in range(4):
        pltpu.sync_copy(ots[r],
                        outT_hbm.at[pl.ds((wid * 4 + r) * N_PAD, N_PAD)])


@functools.lru_cache(maxsize=1)
def _make_gat_sc():
    mesh = plsc.VectorSubcoreMesh(core_axis_name="c", subcore_axis_name="s",
                                  num_cores=NC, num_subcores=NS)
    return pl.kernel(
        _gat_sc_body,
        out_type=[jax.ShapeDtypeStruct((HID * N_PAD,), jnp.float32),
                  jax.ShapeDtypeStruct((NC * E2,), jnp.float32)],
        mesh=mesh,
        scratch_types=[
            pltpu.VMEM((N_PAD,), jnp.float32),
            pltpu.VMEM((N_PAD,), jnp.float32),
            pltpu.VMEM((DROWS, HID), jnp.float32),
            pltpu.VMEM((N_PAD,), jnp.float32),
            pltpu.VMEM((N_PAD,), jnp.float32),
            pltpu.VMEM((N_PAD,), jnp.float32),
            pltpu.VMEM((N_PAD,), jnp.float32),
            pltpu.VMEM((N_PAD,), jnp.float32),
            pltpu.VMEM((N_PAD,), jnp.float32),
            pltpu.VMEM((N_PAD,), jnp.float32),
            pltpu.VMEM((N_PAD,), jnp.float32),
            pltpu.VMEM((2, CHUNK), jnp.int32),
            pltpu.VMEM((2, CHUNK), jnp.int32),
            pltpu.VMEM((2, CHUNK), jnp.float32),
            pltpu.VMEM((LANES,), jnp.float32),
            pltpu.VMEM((DROWS,), jnp.int32),
            pltpu.VMEM_SHARED((DROWS, HID), jnp.float32),
            pltpu.SemaphoreType.DMA((2,)),
        ],
        compiler_params=pltpu.CompilerParams(needs_layout_passes=False),
    )


def _dec_body(tx_ref, rTh_ref, rTc_ref, w_ref, ghb_ref, gcb_ref, db_ref,
              out_ref):
    w = w_ref[...]
    w1, w2, w3 = w[:, 0:HID], w[:, HID:2 * HID], w[:, 2 * HID:3 * HID]
    y = lax.dot_general(w1, tx_ref[...], (((1,), (1,)), ((), ())),
                        preferred_element_type=jnp.float32)
    y = y + lax.dot_general(w2, rTh_ref[...], (((1,), (0,)), ((), ())),
                            preferred_element_type=jnp.float32)
    y = y + lax.dot_general(w3, rTc_ref[...], (((1,), (0,)), ((), ())),
                            preferred_element_type=jnp.float32)
    cb = (jnp.sum(w2 * ghb_ref[...]) + jnp.sum(w3 * gcb_ref[...])
          + db_ref[0, 0])
    out_ref[...] = y + cb


def _run_dec(tx, rTh, rTc, w, ghb, gcb, db):
    return pl.pallas_call(
        _dec_body,
        grid=(NB,),
        in_specs=[
            pl.BlockSpec((BN, HID), lambda j: (j, 0)),
            pl.BlockSpec((HID, BN), lambda j: (0, j)),
            pl.BlockSpec((HID, BN), lambda j: (0, j)),
            pl.BlockSpec((1, 3 * HID), lambda j: (0, 0)),
            pl.BlockSpec((1, HID), lambda j: (0, 0)),
            pl.BlockSpec((1, HID), lambda j: (0, 0)),
            pl.BlockSpec((1, 1), lambda j: (0, 0)),
        ],
        out_specs=pl.BlockSpec((1, BN), lambda j: (0, j)),
        out_shape=jax.ShapeDtypeStruct((1, N_PAD), jnp.float32),
    )(tx, rTh, rTc, w, ghb, gcb, db)


def _prep_edges(ei):
    s, d = ei[0].astype(jnp.int32), ei[1].astype(jnp.int32)
    is_loop = s == d
    pad = jnp.int32(N_NODES)
    s = jnp.where(is_loop, pad, s)
    d = jnp.where(is_loop, pad, d)
    ar = jnp.arange(N_NODES, dtype=jnp.int32)
    npad_e = E2 - E_TOT
    fill = jnp.full((npad_e,), pad, jnp.int32)
    src = jnp.concatenate([s, ar, fill])
    dst = jnp.concatenate([d, ar, fill])
    return src, dst


def kernel(x, edge_index_hetero, edge_index,
           lstm_W_ih, lstm_W_hh, lstm_b_ih, lstm_b_hh,
           gh_W, gh_att_src, gh_att_dst, gh_bias,
           gc_W, gc_att_src, gc_att_dst, gc_bias,
           dec_W, dec_b):
    xp = jnp.pad(x.reshape(N_NODES, SEQ * INP),
                 ((0, N_PAD - N_NODES), (0, 0)))
    bias = (lstm_b_ih + lstm_b_hh).reshape(1, 4 * HID)
    tx = _run_lstm(xp, lstm_W_ih, lstm_W_hh, bias)

    att = jnp.concatenate([
        gh_att_src.reshape(1, HID), gh_att_dst.reshape(1, HID),
        gc_att_src.reshape(1, HID), gc_att_dst.reshape(1, HID),
        jnp.zeros((4, HID), jnp.float32)], axis=0)
    xlTh, xlTc, avec, mvec = _run_proj(tx, gh_W, gc_W, att)

    m_h = jnp.full((LANES,), mvec[0, 0] + mvec[1, 0], jnp.float32)
    m_c = jnp.full((LANES,), mvec[2, 0] + mvec[3, 0], jnp.float32)

    src_h, dst_h = _prep_edges(edge_index_hetero)
    src_c, dst_c = _prep_edges(edge_index)

    gat = _make_gat_sc()
    rTh, _ = gat(avec[0], avec[1], xlTh.reshape(-1), src_h, dst_h, m_h)
    rTc, _ = gat(avec[2], avec[3], xlTc.reshape(-1), src_c, dst_c, m_c)
    rTh = rTh.reshape(HID, N_PAD)
    rTc = rTc.reshape(HID, N_PAD)

    out = _run_dec(tx, rTh, rTc, dec_W.reshape(1, 3 * HID),
                   gh_bias.reshape(1, HID), gc_bias.reshape(1, HID),
                   dec_b.reshape(1, 1))
    return out.reshape(N_PAD, 1)[:N_NODES]
